# baseline (speedup 1.0000x reference)
import functools

import jax
import jax.numpy as jnp
from jax import lax
from jax.experimental import pallas as pl
from jax.experimental.pallas import tpu as pltpu
from jax.experimental.pallas import tpu_sc as plsc

BN = 512
KC = 48
SC_NW = 32
SC_L = 128
SC_G = 8
NPAD_ALIGN = 2048


def _sc_gather(table, idx2d):
    n_idx_rows, _ = idx2d.shape
    d = table.shape[1]
    n_rows = table.shape[0]
    dt = table.dtype
    m = n_idx_rows * SC_L
    idx_flat = idx2d.reshape(m)
    per_w = m // SC_NW
    ch = SC_G * SC_L
    n_super = per_w // ch
    rows_per_tile = n_rows // 16
    mesh = plsc.VectorSubcoreMesh(core_axis_name="c", subcore_axis_name="s")

    @functools.partial(
        pl.kernel,
        mesh=mesh,
        out_type=jax.ShapeDtypeStruct((m, d), dt),
        scratch_types=[
            pltpu.VMEM((ch,), jnp.int32),
            pltpu.VMEM((ch,), jnp.int32),
            pltpu.VMEM((ch, d), dt),
            pltpu.VMEM((ch, d), dt),
            pltpu.VMEM_SHARED((n_rows, d), dt),
            pltpu.SemaphoreType.DMA,
            pltpu.SemaphoreType.DMA,
            pltpu.SemaphoreType.DMA,
        ],
        compiler_params=pltpu.CompilerParams(use_tc_tiling_on_sc=False),
    )
    def gather_kernel(idx_hbm, table_hbm, out_hbm,
                      idx_v0, idx_v1, rows_v0, rows_v1, table_s,
                      gat_sem, out_sem0, out_sem1):
        wid = lax.axis_index("s") * 2 + lax.axis_index("c")
        base = wid * per_w

        sid = lax.axis_index("s")
        pltpu.sync_copy(
            table_hbm.at[pl.ds(sid * rows_per_tile, rows_per_tile)],
            table_s.at[pl.ds(sid * rows_per_tile, rows_per_tile)],
        )
        plsc.subcore_barrier()

        def chunk(s, idx_v, rows_v, out_sem):
            off = base + s * ch

            @pl.when(s >= 2)
            def _():
                pltpu.make_async_copy(
                    rows_v, out_hbm.at[pl.ds(off, ch)], out_sem
                ).wait()

            pltpu.sync_copy(idx_hbm.at[pl.ds(off, ch)], idx_v)
            pltpu.async_copy(table_s.at[idx_v], rows_v, gat_sem).wait()
            pltpu.async_copy(rows_v, out_hbm.at[pl.ds(off, ch)], out_sem)

        def body(s, carry):
            @pl.when(s % 2 == 0)
            def _():
                chunk(s, idx_v0, rows_v0, out_sem0)

            @pl.when(s % 2 == 1)
            def _():
                chunk(s, idx_v1, rows_v1, out_sem1)

            return carry

        lax.fori_loop(0, n_super, body, 0)
        if n_super >= 2:
            pltpu.make_async_copy(
                rows_v0, out_hbm.at[pl.ds(base, ch)], out_sem0).wait()
            pltpu.make_async_copy(
                rows_v1, out_hbm.at[pl.ds(base, ch)], out_sem1).wait()

    return gather_kernel(idx_flat, table)


def _lstm_chunk(gx, h, c, counts2, scal, wcat_t, bias, hd):
    n_pad = h.shape[0]
    dp = gx.shape[2]
    gd = wcat_t.shape[1]

    def body(scal_ref, gx_ref, h_ref, c_ref, cnt_ref, w_ref, b_ref,
             ho_ref, co_ref):
        k0 = scal_ref[0]
        k_tot = scal_ref[1]
        keff = jnp.minimum(k_tot - k0, KC)
        cnt = cnt_ref[...]

        def step(k, carry):
            h_c, c_c = carry
            xt = gx_ref[k].astype(jnp.float32)
            xc = jnp.concatenate([xt, h_c], axis=1)
            z = jnp.dot(xc, w_ref[...],
                        preferred_element_type=jnp.float32) + b_ref[...]
            gi = z[:, 0 * hd:1 * hd]
            gf = z[:, 1 * hd:2 * hd]
            gg = z[:, 2 * hd:3 * hd]
            go = z[:, 3 * hd:4 * hd]
            c_n = jax.nn.sigmoid(gf) * c_c + jax.nn.sigmoid(gi) * jnp.tanh(gg)
            h_n = jax.nn.sigmoid(go) * jnp.tanh(c_n)
            mask = (k0 + k) < cnt
            return jnp.where(mask, h_n, h_c), jnp.where(mask, c_n, c_c)

        hf, cf = lax.fori_loop(0, keff, step, (h_ref[...], c_ref[...]))
        ho_ref[...] = hf
        co_ref[...] = cf

    grid = (n_pad // BN,)
    return pl.pallas_call(
        body,
        grid=grid,
        in_specs=[
            pl.BlockSpec(memory_space=pltpu.SMEM),
            pl.BlockSpec((KC, BN, dp), lambda nb: (0, nb, 0)),
            pl.BlockSpec((BN, hd), lambda nb: (nb, 0)),
            pl.BlockSpec((BN, hd), lambda nb: (nb, 0)),
            pl.BlockSpec((BN, 1), lambda nb: (nb, 0)),
            pl.BlockSpec((dp + hd, gd), lambda nb: (0, 0)),
            pl.BlockSpec((1, gd), lambda nb: (0, 0)),
        ],
        out_specs=[
            pl.BlockSpec((BN, hd), lambda nb: (nb, 0)),
            pl.BlockSpec((BN, hd), lambda nb: (nb, 0)),
        ],
        out_shape=[
            jax.ShapeDtypeStruct((n_pad, hd), jnp.float32),
            jax.ShapeDtypeStruct((n_pad, hd), jnp.float32),
        ],
    )(scal, gx, h, c, counts2, wcat_t, bias)


def _dense1(xf, h1, wd1, b1):
    n_pad = xf.shape[0]

    def body(xf_ref, h_ref, w_ref, b_ref, o_ref):
        xc = jnp.concatenate([xf_ref[...], h_ref[...]], axis=1)
        z = jnp.dot(xc, w_ref[...],
                    preferred_element_type=jnp.float32) + b_ref[...]
        o_ref[...] = jnp.maximum(z, 0.0)

    return pl.pallas_call(
        body,
        grid=(n_pad // BN,),
        in_specs=[
            pl.BlockSpec((BN, xf.shape[1]), lambda nb: (nb, 0)),
            pl.BlockSpec((BN, h1.shape[1]), lambda nb: (nb, 0)),
            pl.BlockSpec(wd1.shape, lambda nb: (0, 0)),
            pl.BlockSpec(b1.shape, lambda nb: (0, 0)),
        ],
        out_specs=pl.BlockSpec((BN, 32), lambda nb: (nb, 0)),
        out_shape=jax.ShapeDtypeStruct((n_pad, 32), jnp.float32),
    )(xf, h1, wd1, b1)


def _dense2_head(x1, h2, wd2, b2, fc1_t, fc1_b, fc2_t, fc2_b):
    n_pad = x1.shape[0]

    def body(x_ref, h_ref, w_ref, b_ref, w1_ref, b1_ref, w2_ref, b2_ref,
             o_ref):
        xc = jnp.concatenate([x_ref[...], h_ref[...]], axis=1)
        z = jnp.dot(xc, w_ref[...],
                    preferred_element_type=jnp.float32) + b_ref[...]
        x2 = jnp.maximum(z, 0.0)
        t = jnp.dot(x2, w1_ref[...],
                    preferred_element_type=jnp.float32) + b1_ref[...]
        t = jnp.where(t > 0, t, 0.01 * t)
        y = jnp.dot(t, w2_ref[...],
                    preferred_element_type=jnp.float32) + b2_ref[0, 0]
        o_ref[...] = y

    return pl.pallas_call(
        body,
        grid=(n_pad // BN,),
        in_specs=[
            pl.BlockSpec((BN, 32), lambda nb: (nb, 0)),
            pl.BlockSpec((BN, 32), lambda nb: (nb, 0)),
            pl.BlockSpec(wd2.shape, lambda nb: (0, 0)),
            pl.BlockSpec(b2.shape, lambda nb: (0, 0)),
            pl.BlockSpec(fc1_t.shape, lambda nb: (0, 0)),
            pl.BlockSpec(fc1_b.shape, lambda nb: (0, 0)),
            pl.BlockSpec(fc2_t.shape, lambda nb: (0, 0)),
            pl.BlockSpec((1, 1), lambda nb: (0, 0), memory_space=pltpu.SMEM),
        ],
        out_specs=pl.BlockSpec((BN, 8), lambda nb: (nb, 0)),
        out_shape=jax.ShapeDtypeStruct((n_pad, 8), jnp.float32),
    )(x1, h2, wd2, b2, fc1_t, fc1_b, fc2_t, fc2_b)


def _layer(table_p, counts2, rank, dst_s, src_s, k_max, wcat_t, bias, hd):
    n_pad = table_p.shape[0]
    m = KC * n_pad
    h0 = jnp.zeros((n_pad, hd), jnp.float32)

    def cond(s):
        return s[0] < k_max

    def body(s):
        k0, h, c = s
        rel = rank - k0
        init = jnp.broadcast_to(
            jnp.arange(n_pad, dtype=jnp.int32)[None, :], (KC, n_pad)
        ).reshape(m)
        pos = rel * n_pad + dst_s
        nbr = init.at[pos].add(src_s - dst_s, mode="drop",
                               unique_indices=True)
        gx = _sc_gather(table_p, nbr.reshape(m // SC_L, SC_L))
        gx = gx.reshape(KC, n_pad, table_p.shape[1])
        scal = jnp.stack([k0, k_max]).astype(jnp.int32)
        h, c = _lstm_chunk(gx, h, c, counts2, scal, wcat_t, bias, hd)
        return k0 + KC, h, c

    _, h, _ = lax.while_loop(cond, body, (jnp.int32(0), h0, h0))
    return h


def _regate(wt, hd, hp):
    if hd == hp:
        return wt
    parts = [
        jnp.pad(wt[:, g * hd:(g + 1) * hd], ((0, 0), (0, hp - hd)))
        for g in range(4)
    ]
    return jnp.concatenate(parts, axis=1)


def _pad_rows(a, rows):
    return jnp.pad(a, ((0, rows - a.shape[0]), (0, 0)))


def kernel(node_features, edge_index,
           l1_W_ih, l1_W_hh, l1_b_ih, l1_b_hh, l1_fc_self_w, l1_fc_self_b,
           l1_fc_neigh_w,
           l2_W_ih, l2_W_hh, l2_b_ih, l2_b_hh, l2_fc_self_w, l2_fc_self_b,
           l2_fc_neigh_w,
           fc1_w, fc1_b, fc2_w, fc2_b):
    n = node_features.shape[0]
    n_pad = ((n + NPAD_ALIGN - 1) // NPAD_ALIGN) * NPAD_ALIGN

    src = edge_index[0]
    dst = edge_index[1]
    counts = jnp.zeros((n_pad,), jnp.int32).at[dst].add(1)
    order = jnp.argsort(dst, stable=True)
    dst_s = dst[order]
    src_s = src[order]
    starts = jnp.cumsum(counts) - counts
    rank = jnp.arange(dst.shape[0], dtype=jnp.int32) - starts[dst_s]
    k_max = jnp.max(counts)
    counts2 = counts[:, None]

    xf = _pad_rows(jnp.pad(node_features, ((0, 0), (0, 11))), n_pad)
    w1 = jnp.concatenate([
        _pad_rows(_regate(l1_W_ih.T, 5, 8), 16),
        _pad_rows(_regate(l1_W_hh.T, 5, 8), 8),
    ])
    b1 = _regate((l1_b_ih + l1_b_hh)[None, :], 5, 8)
    h1 = _layer(xf, counts2, rank, dst_s, src_s, k_max, w1, b1, 8)

    wd1 = jnp.concatenate([
        _pad_rows(l1_fc_self_w.T, 16),
        _pad_rows(l1_fc_neigh_w.T, 8),
    ])
    x1 = _dense1(xf, h1, wd1, l1_fc_self_b[None, :])

    w2 = jnp.concatenate([l2_W_ih.T, l2_W_hh.T])
    b2 = (l2_b_ih + l2_b_hh)[None, :]
    h2 = _layer(x1.astype(jnp.bfloat16), counts2, rank, dst_s, src_s, k_max,
                w2, b2, 32)

    wd2 = jnp.concatenate([l2_fc_self_w.T, l2_fc_neigh_w.T])
    y = _dense2_head(
        x1, h2, wd2, l2_fc_self_b[None, :],
        fc1_w.T, fc1_b[None, :],
        jnp.pad(fc2_w.T, ((0, 0), (0, 7))), fc2_b.reshape(1, 1),
    )
    return y[:n, :1]


# speedup vs baseline: 1.0097x; 1.0097x over previous
import functools

import jax
import jax.numpy as jnp
from jax import lax
from jax.experimental import pallas as pl
from jax.experimental.pallas import tpu as pltpu
from jax.experimental.pallas import tpu_sc as plsc

BN = 512
KC = 48
SC_NW = 32
SC_L = 128
SC_G = 8
NPAD_ALIGN = 2048


def _sc_gather(table, idx2d):
    n_idx_rows, _ = idx2d.shape
    d = table.shape[1]
    n_rows = table.shape[0]
    dt = table.dtype
    m = n_idx_rows * SC_L
    idx_flat = idx2d.reshape(m)
    per_w = m // SC_NW
    ch = SC_G * SC_L
    n_super = per_w // ch
    rows_per_tile = n_rows // 16
    mesh = plsc.VectorSubcoreMesh(core_axis_name="c", subcore_axis_name="s")

    @functools.partial(
        pl.kernel,
        mesh=mesh,
        out_type=jax.ShapeDtypeStruct((m, d), dt),
        scratch_types=[
            pltpu.VMEM((ch,), jnp.int32),
            pltpu.VMEM((ch,), jnp.int32),
            pltpu.VMEM((ch, d), dt),
            pltpu.VMEM((ch, d), dt),
            pltpu.VMEM_SHARED((n_rows, d), dt),
            pltpu.SemaphoreType.DMA,
            pltpu.SemaphoreType.DMA,
            pltpu.SemaphoreType.DMA,
        ],
        compiler_params=pltpu.CompilerParams(use_tc_tiling_on_sc=False),
    )
    def gather_kernel(idx_hbm, table_hbm, out_hbm,
                      idx_v0, idx_v1, rows_v0, rows_v1, table_s,
                      gat_sem, out_sem0, out_sem1):
        wid = lax.axis_index("s") * 2 + lax.axis_index("c")
        base = wid * per_w

        sid = lax.axis_index("s")
        pltpu.sync_copy(
            table_hbm.at[pl.ds(sid * rows_per_tile, rows_per_tile)],
            table_s.at[pl.ds(sid * rows_per_tile, rows_per_tile)],
        )
        plsc.subcore_barrier()

        def chunk(s, idx_v, rows_v, out_sem):
            off = base + s * ch

            @pl.when(s >= 2)
            def _():
                pltpu.make_async_copy(
                    rows_v, out_hbm.at[pl.ds(off, ch)], out_sem
                ).wait()

            pltpu.sync_copy(idx_hbm.at[pl.ds(off, ch)], idx_v)
            pltpu.async_copy(table_s.at[idx_v], rows_v, gat_sem).wait()
            pltpu.async_copy(rows_v, out_hbm.at[pl.ds(off, ch)], out_sem)

        def body(s, carry):
            @pl.when(s % 2 == 0)
            def _():
                chunk(s, idx_v0, rows_v0, out_sem0)

            @pl.when(s % 2 == 1)
            def _():
                chunk(s, idx_v1, rows_v1, out_sem1)

            return carry

        lax.fori_loop(0, n_super, body, 0)
        if n_super >= 2:
            pltpu.make_async_copy(
                rows_v0, out_hbm.at[pl.ds(base, ch)], out_sem0).wait()
            pltpu.make_async_copy(
                rows_v1, out_hbm.at[pl.ds(base, ch)], out_sem1).wait()

    return gather_kernel(idx_flat, table)


def _lstm_chunk(gx, h, c, counts2, scal, wcat_t, bias, hd):
    n_pad = h.shape[0]
    dp = gx.shape[2]
    gd = wcat_t.shape[1]

    def body(scal_ref, gx_ref, h_ref, c_ref, cnt_ref, w_ref, b_ref,
             ho_ref, co_ref):
        k0 = scal_ref[0]
        k_tot = scal_ref[1]
        keff = jnp.minimum(k_tot - k0, KC)
        cnt = cnt_ref[...]

        def step(k, carry):
            h_c, c_c = carry
            xt = gx_ref[k].astype(jnp.bfloat16)
            xc = jnp.concatenate([xt, h_c.astype(jnp.bfloat16)], axis=1)
            z = jnp.dot(xc, w_ref[...],
                        preferred_element_type=jnp.float32) + b_ref[...]
            gi = z[:, 0 * hd:1 * hd]
            gf = z[:, 1 * hd:2 * hd]
            gg = z[:, 2 * hd:3 * hd]
            go = z[:, 3 * hd:4 * hd]
            c_n = jax.nn.sigmoid(gf) * c_c + jax.nn.sigmoid(gi) * jnp.tanh(gg)
            h_n = jax.nn.sigmoid(go) * jnp.tanh(c_n)
            mask = (k0 + k) < cnt
            return jnp.where(mask, h_n, h_c), jnp.where(mask, c_n, c_c)

        hf, cf = lax.fori_loop(0, keff, step, (h_ref[...], c_ref[...]))
        ho_ref[...] = hf
        co_ref[...] = cf

    grid = (n_pad // BN,)
    return pl.pallas_call(
        body,
        grid=grid,
        in_specs=[
            pl.BlockSpec(memory_space=pltpu.SMEM),
            pl.BlockSpec((KC, BN, dp), lambda nb: (0, nb, 0)),
            pl.BlockSpec((BN, hd), lambda nb: (nb, 0)),
            pl.BlockSpec((BN, hd), lambda nb: (nb, 0)),
            pl.BlockSpec((BN, 1), lambda nb: (nb, 0)),
            pl.BlockSpec((dp + hd, gd), lambda nb: (0, 0)),
            pl.BlockSpec((1, gd), lambda nb: (0, 0)),
        ],
        out_specs=[
            pl.BlockSpec((BN, hd), lambda nb: (nb, 0)),
            pl.BlockSpec((BN, hd), lambda nb: (nb, 0)),
        ],
        out_shape=[
            jax.ShapeDtypeStruct((n_pad, hd), jnp.float32),
            jax.ShapeDtypeStruct((n_pad, hd), jnp.float32),
        ],
    )(scal, gx, h, c, counts2, wcat_t, bias)


def _dense1(xf, h1, wd1, b1):
    n_pad = xf.shape[0]

    def body(xf_ref, h_ref, w_ref, b_ref, o_ref):
        xc = jnp.concatenate([xf_ref[...], h_ref[...]], axis=1)
        z = jnp.dot(xc, w_ref[...],
                    preferred_element_type=jnp.float32) + b_ref[...]
        o_ref[...] = jnp.maximum(z, 0.0)

    return pl.pallas_call(
        body,
        grid=(n_pad // BN,),
        in_specs=[
            pl.BlockSpec((BN, xf.shape[1]), lambda nb: (nb, 0)),
            pl.BlockSpec((BN, h1.shape[1]), lambda nb: (nb, 0)),
            pl.BlockSpec(wd1.shape, lambda nb: (0, 0)),
            pl.BlockSpec(b1.shape, lambda nb: (0, 0)),
        ],
        out_specs=pl.BlockSpec((BN, 32), lambda nb: (nb, 0)),
        out_shape=jax.ShapeDtypeStruct((n_pad, 32), jnp.float32),
    )(xf, h1, wd1, b1)


def _dense2_head(x1, h2, wd2, b2, fc1_t, fc1_b, fc2_t, fc2_b):
    n_pad = x1.shape[0]

    def body(x_ref, h_ref, w_ref, b_ref, w1_ref, b1_ref, w2_ref, b2_ref,
             o_ref):
        xc = jnp.concatenate([x_ref[...], h_ref[...]], axis=1)
        z = jnp.dot(xc, w_ref[...],
                    preferred_element_type=jnp.float32) + b_ref[...]
        x2 = jnp.maximum(z, 0.0)
        t = jnp.dot(x2, w1_ref[...],
                    preferred_element_type=jnp.float32) + b1_ref[...]
        t = jnp.where(t > 0, t, 0.01 * t)
        y = jnp.dot(t, w2_ref[...],
                    preferred_element_type=jnp.float32) + b2_ref[0, 0]
        o_ref[...] = y

    return pl.pallas_call(
        body,
        grid=(n_pad // BN,),
        in_specs=[
            pl.BlockSpec((BN, 32), lambda nb: (nb, 0)),
            pl.BlockSpec((BN, 32), lambda nb: (nb, 0)),
            pl.BlockSpec(wd2.shape, lambda nb: (0, 0)),
            pl.BlockSpec(b2.shape, lambda nb: (0, 0)),
            pl.BlockSpec(fc1_t.shape, lambda nb: (0, 0)),
            pl.BlockSpec(fc1_b.shape, lambda nb: (0, 0)),
            pl.BlockSpec(fc2_t.shape, lambda nb: (0, 0)),
            pl.BlockSpec((1, 1), lambda nb: (0, 0), memory_space=pltpu.SMEM),
        ],
        out_specs=pl.BlockSpec((BN, 8), lambda nb: (nb, 0)),
        out_shape=jax.ShapeDtypeStruct((n_pad, 8), jnp.float32),
    )(x1, h2, wd2, b2, fc1_t, fc1_b, fc2_t, fc2_b)


def _layer(table_p, counts2, rank, dst_s, src_s, k_max, wcat_t, bias, hd):
    n_pad = table_p.shape[0]
    m = KC * n_pad
    h0 = jnp.zeros((n_pad, hd), jnp.float32)

    def cond(s):
        return s[0] < k_max

    def body(s):
        k0, h, c = s
        rel = rank - k0
        init = jnp.broadcast_to(
            jnp.arange(n_pad, dtype=jnp.int32)[None, :], (KC, n_pad)
        ).reshape(m)
        pos = rel * n_pad + dst_s
        nbr = init.at[pos].add(src_s - dst_s, mode="drop",
                               unique_indices=True)
        gx = _sc_gather(table_p, nbr.reshape(m // SC_L, SC_L))
        gx = gx.reshape(KC, n_pad, table_p.shape[1])
        scal = jnp.stack([k0, k_max]).astype(jnp.int32)
        h, c = _lstm_chunk(gx, h, c, counts2, scal, wcat_t, bias, hd)
        return k0 + KC, h, c

    _, h, _ = lax.while_loop(cond, body, (jnp.int32(0), h0, h0))
    return h


def _regate(wt, hd, hp):
    if hd == hp:
        return wt
    parts = [
        jnp.pad(wt[:, g * hd:(g + 1) * hd], ((0, 0), (0, hp - hd)))
        for g in range(4)
    ]
    return jnp.concatenate(parts, axis=1)


def _pad_rows(a, rows):
    return jnp.pad(a, ((0, rows - a.shape[0]), (0, 0)))


def kernel(node_features, edge_index,
           l1_W_ih, l1_W_hh, l1_b_ih, l1_b_hh, l1_fc_self_w, l1_fc_self_b,
           l1_fc_neigh_w,
           l2_W_ih, l2_W_hh, l2_b_ih, l2_b_hh, l2_fc_self_w, l2_fc_self_b,
           l2_fc_neigh_w,
           fc1_w, fc1_b, fc2_w, fc2_b):
    n = node_features.shape[0]
    n_pad = ((n + NPAD_ALIGN - 1) // NPAD_ALIGN) * NPAD_ALIGN

    src = edge_index[0]
    dst = edge_index[1]
    counts = jnp.zeros((n_pad,), jnp.int32).at[dst].add(1)
    order = jnp.argsort(dst, stable=True)
    dst_s = dst[order]
    src_s = src[order]
    starts = jnp.cumsum(counts) - counts
    rank = jnp.arange(dst.shape[0], dtype=jnp.int32) - starts[dst_s]
    k_max = jnp.max(counts)
    counts2 = counts[:, None]

    xf = _pad_rows(jnp.pad(node_features, ((0, 0), (0, 11))), n_pad)
    w1 = jnp.concatenate([
        _pad_rows(_regate(l1_W_ih.T, 5, 8), 16),
        _pad_rows(_regate(l1_W_hh.T, 5, 8), 8),
    ]).astype(jnp.bfloat16)
    b1 = _regate((l1_b_ih + l1_b_hh)[None, :], 5, 8)
    h1 = _layer(xf, counts2, rank, dst_s, src_s, k_max, w1, b1, 8)

    wd1 = jnp.concatenate([
        _pad_rows(l1_fc_self_w.T, 16),
        _pad_rows(l1_fc_neigh_w.T, 8),
    ])
    x1 = _dense1(xf, h1, wd1, l1_fc_self_b[None, :])

    w2 = jnp.concatenate([l2_W_ih.T, l2_W_hh.T]).astype(jnp.bfloat16)
    b2 = (l2_b_ih + l2_b_hh)[None, :]
    h2 = _layer(x1.astype(jnp.bfloat16), counts2, rank, dst_s, src_s, k_max,
                w2, b2, 32)

    wd2 = jnp.concatenate([l2_fc_self_w.T, l2_fc_neigh_w.T])
    y = _dense2_head(
        x1, h2, wd2, l2_fc_self_b[None, :],
        fc1_w.T, fc1_b[None, :],
        jnp.pad(fc2_w.T, ((0, 0), (0, 7))), fc2_b.reshape(1, 1),
    )
    return y[:n, :1]
